# baseline (speedup 1.0000x reference)
import functools

import jax
import jax.numpy as jnp
from jax import lax
from jax.experimental import pallas as pl
from jax.experimental.pallas import tpu as pltpu
from jax.experimental.pallas import tpu_sc as plsc

N = 100000
H = 128
OUT = 128
NG = 1024
STEPS = 6

NC = 2
NS = 16
NW = NC * NS
GPW = NG // NW
T = 256
U = 4
NEG = -1e30
L = 16
KV = H // L
MAXT = (N + T - 1) // T + 1
MAXT = (MAXT + 15) // 16 * 16


def _splat_i32(x):
    return jnp.broadcast_to(jnp.int32(x) if isinstance(x, int) else x.astype(jnp.int32), (L,))


def _extract_i32(ref, idx_scalar):
    idx = jnp.int32(idx_scalar) if isinstance(idx_scalar, int) else idx_scalar.astype(jnp.int32)
    base = idx // L * L
    w = ref[pl.ds(base, L)]
    lane = _splat_i32(idx - base)
    sel = jnp.where(lax.iota(jnp.int32, L) == lane, w, jnp.int32(-2147483648))
    return jnp.max(sel)


def _hsum_splat(acc):
    return jnp.broadcast_to(jnp.sum(acc), (L,))


def _make_readout():
    mesh = plsc.VectorSubcoreMesh(core_axis_name="c", subcore_axis_name="s")

    @functools.partial(
        pl.kernel,
        mesh=mesh,
        compiler_params=pltpu.CompilerParams(needs_layout_passes=False),
        out_type=jax.ShapeDtypeStruct((NG, H), jnp.float32),
        scratch_types=[
            pltpu.VMEM((GPW + L,), jnp.int32),
            pltpu.VMEM((GPW, H), jnp.float32),
            pltpu.VMEM((GPW, H), jnp.float32),
            pltpu.VMEM((2 * T, H), jnp.float32),
            pltpu.VMEM((MAXT,), jnp.int32),
            pltpu.VMEM((MAXT,), jnp.int32),
        ],
    )
    def readout(ne_hbm, starts_hbm, h_hbm, jlo_hbm, nseg_hbm, r_hbm,
                starts_v, h_v, r_v, buf, jlo_v, nseg_v):
        wid = lax.axis_index("s") * NC + lax.axis_index("c")
        g0 = wid * GPW
        pltpu.sync_copy(starts_hbm.at[pl.ds(g0, GPW + L)], starts_v)
        pltpu.sync_copy(h_hbm.at[pl.ds(g0, GPW)], h_v)
        pltpu.sync_copy(jlo_hbm.at[pl.ds(wid * MAXT, MAXT)], jlo_v)
        pltpu.sync_copy(nseg_hbm.at[pl.ds(wid * MAXT, MAXT)], nseg_v)

        zerov = jnp.zeros((L,), jnp.float32)

        def zero_body(j, _):
            for k in range(KV):
                r_v[j, pl.ds(L * k, L)] = zerov
            return 0

        lax.fori_loop(0, GPW, zero_body, 0)

        LO = _extract_i32(starts_v, 0)
        HI = _extract_i32(starts_v, GPW)
        LO8 = LO // 8 * 8
        nt = (HI - LO8 + (T - 1)) // T
        nt1 = jnp.maximum(nt, 1)
        nt2 = nt1 + (nt1 % 2)
        last = nt2 - 1

        def _fb(t):
            return jnp.minimum(LO8 + t * T, N - T)

        def process_tile(t, slot, carry):
            tb = LO8 + t * T
            fb = _fb(t)
            jlo = _extract_i32(jlo_v, t)
            njs = _extract_i32(nseg_v, t)
            tb_v = _splat_i32(tb)
            off0 = slot + (tb - fb)
            hi_clamp = slot + (T - 1)

            def seg_body(k, sc):
                m_v, s_v = sc[0], sc[1]
                v_list = list(sc[2:])
                j = jlo + k
                lo = _extract_i32(starts_v, j)
                hi = _extract_i32(starts_v, j + 1)
                hvecs = [h_v[j, pl.ds(L * kk, L)] for kk in range(KV)]
                isnew = _splat_i32(lo) >= tb_v
                m_v = jnp.where(isnew, NEG, m_v)
                s_v = jnp.where(isnew, 0.0, s_v)
                v_list = [jnp.where(isnew, 0.0, vk) for vk in v_list]
                s_lo = jnp.maximum(lo, tb)
                s_hi = jnp.minimum(hi, tb + T)
                ngrp = (s_hi - s_lo + (U - 1)) // U
                off = off0 + (s_lo - tb)
                rend_v = _splat_i32(s_hi)

                def group_body(g, gc):
                    m_v, s_v = gc[0], gc[1]
                    v_list = list(gc[2:])
                    base = off + g * U
                    rows = []
                    es = []
                    for u in range(U):
                        iloc = jnp.minimum(base + u, hi_clamp)
                        rvs = [buf[iloc, pl.ds(L * kk, L)] for kk in range(KV)]
                        acc = rvs[0] * hvecs[0]
                        for kk in range(1, KV):
                            acc = acc + rvs[kk] * hvecs[kk]
                        e_u = _hsum_splat(acc)
                        valid = _splat_i32(s_lo + g * U + u) < rend_v
                        es.append(jnp.where(valid, e_u, NEG))
                        rows.append(rvs)
                    m_new = m_v
                    for u in range(U):
                        m_new = jnp.maximum(m_new, es[u])
                    alpha = jnp.exp(m_v - m_new)
                    ws = [jnp.exp(es[u] - m_new) for u in range(U)]
                    wsum = ws[0]
                    for u in range(1, U):
                        wsum = wsum + ws[u]
                    s_v = s_v * alpha + wsum
                    new_v = []
                    for kk in range(KV):
                        vk = v_list[kk] * alpha
                        for u in range(U):
                            vk = vk + ws[u] * rows[u][kk]
                        new_v.append(vk)
                    return (m_new, s_v, *new_v)

                res = lax.fori_loop(0, ngrp, group_body, (m_v, s_v, *v_list))
                s_fin = res[1]
                denom = s_fin + jnp.float32(1e-16)
                for kk in range(KV):
                    r_v[j, pl.ds(L * kk, L)] = res[2 + kk] / denom
                return res

            return lax.fori_loop(0, njs, seg_body, carry)

        def pair_body(i, carry):
            t0 = 2 * i
            t1 = t0 + 1
            pltpu.sync_copy(ne_hbm.at[pl.ds(_fb(t0), T)], buf.at[pl.ds(0, T)])
            carry = process_tile(t0, 0, carry)
            pltpu.sync_copy(ne_hbm.at[pl.ds(_fb(t1), T)], buf.at[pl.ds(T, T)])
            carry = process_tile(t1, T, carry)
            return carry

        init = (jnp.full((L,), NEG, jnp.float32), zerov, *([zerov] * KV))
        lax.fori_loop(0, nt2 // 2, pair_body, init)

        pltpu.sync_copy(r_v, r_hbm.at[pl.ds(g0, GPW)])

    return readout


_READOUT = _make_readout()


def _lstm_body(h_ref, r_ref, c_ref, wh_ref, wr_ref, b_ref, h_out, c_out):
    gates = (
        jnp.dot(h_ref[...], wh_ref[...], preferred_element_type=jnp.float32)
        + jnp.dot(r_ref[...], wr_ref[...], preferred_element_type=jnp.float32)
        + b_ref[...]
    )
    i = jax.nn.sigmoid(gates[:, :H])
    f = jax.nn.sigmoid(gates[:, H:2 * H])
    g = jnp.tanh(gates[:, 2 * H:3 * H])
    o = jax.nn.sigmoid(gates[:, 3 * H:4 * H])
    c_new = f * c_ref[...] + i * g
    h_out[...] = o * jnp.tanh(c_new)
    c_out[...] = c_new


_LSTM = pl.pallas_call(
    _lstm_body,
    out_shape=[
        jax.ShapeDtypeStruct((NG, H), jnp.float32),
        jax.ShapeDtypeStruct((NG, H), jnp.float32),
    ],
)


def _final_body(h_ref, r_ref, c_ref, wh_ref, wr_ref, b_ref,
                w1h_ref, w1r_ref, b1_ref, w2t_ref, b2_ref, out_ref):
    gates = (
        jnp.dot(h_ref[...], wh_ref[...], preferred_element_type=jnp.float32)
        + jnp.dot(r_ref[...], wr_ref[...], preferred_element_type=jnp.float32)
        + b_ref[...]
    )
    i = jax.nn.sigmoid(gates[:, :H])
    f = jax.nn.sigmoid(gates[:, H:2 * H])
    g = jnp.tanh(gates[:, 2 * H:3 * H])
    o = jax.nn.sigmoid(gates[:, 3 * H:4 * H])
    c_new = f * c_ref[...] + i * g
    h_new = o * jnp.tanh(c_new)
    hidden = jax.nn.relu(
        jnp.dot(h_new, w1h_ref[...], preferred_element_type=jnp.float32)
        + jnp.dot(r_ref[...], w1r_ref[...], preferred_element_type=jnp.float32)
        + b1_ref[...]
    )
    out_ref[...] = (
        jnp.dot(hidden, w2t_ref[...], preferred_element_type=jnp.float32)
        + b2_ref[...]
    )


_FINAL = pl.pallas_call(
    _final_body,
    out_shape=jax.ShapeDtypeStruct((NG, OUT), jnp.float32),
)


def kernel(node_embeddings, batch_indices, W_ih, W_hh, b_ih, b_hh, W1, b1, W2, b2):
    bi = batch_indices.astype(jnp.int32)
    starts = jnp.searchsorted(bi, jnp.arange(NG + 1, dtype=jnp.int32)).astype(jnp.int32)
    lo_list = starts[:NG].reshape(NW, GPW)
    hi_list = starts[1:NG + 1].reshape(NW, GPW)
    LOs = starts[jnp.arange(NW) * GPW]
    LO8s = LOs // 8 * 8
    tbs = LO8s[:, None] + jnp.arange(MAXT, dtype=jnp.int32)[None, :] * T
    jlo = jax.vmap(lambda h_l, q: jnp.searchsorted(h_l, q, side="right"))(hi_list, tbs)
    jhi = jax.vmap(lambda l_l, q: jnp.searchsorted(l_l, q, side="left"))(lo_list, tbs + T)
    nseg = (jhi - jlo).astype(jnp.int32).reshape(-1)
    jlo = jlo.astype(jnp.int32).reshape(-1)
    starts_pad = jnp.concatenate([starts, jnp.full((L - 1,), N, jnp.int32)])

    Wh = (W_ih[:, :H] + W_hh).T
    Wr = W_ih[:, H:].T
    b = (b_ih + b_hh)[None, :]
    W1h = W1[:, :H].T
    W1r = W1[:, H:].T
    b1r = b1[None, :]
    W2T = W2.T
    b2r = b2[None, :]

    h = jnp.zeros((NG, H), jnp.float32)
    c = jnp.zeros((NG, H), jnp.float32)
    out = None
    for step in range(STEPS):
        r = _READOUT(node_embeddings, starts_pad, h, jlo, nseg)
        if step < STEPS - 1:
            h, c = _LSTM(h, r, c, Wh, Wr, b)
        else:
            out = _FINAL(h, r, c, Wh, Wr, b, W1h, W1r, b1r, W2T, b2r)
    return out


# speedup vs baseline: 3.3688x; 3.3688x over previous
import functools

import jax
import jax.numpy as jnp
from jax import lax
from jax.experimental import pallas as pl
from jax.experimental.pallas import tpu as pltpu
from jax.experimental.pallas import tpu_sc as plsc

N = 100000
H = 128
OUT = 128
NG = 1024
STEPS = 6

NC = 2
NS = 16
NW = NC * NS
GPW = NG // NW
T = 256
U = 4
NEG = -1e30
L = 16
KV = H // L
MAXT = (N + T - 1) // T + 1
MAXT = (MAXT + 15) // 16 * 16


def _splat_i32(x):
    return jnp.broadcast_to(jnp.int32(x) if isinstance(x, int) else x.astype(jnp.int32), (L,))


def _extract_i32(ref, idx_scalar):
    idx = jnp.int32(idx_scalar) if isinstance(idx_scalar, int) else idx_scalar.astype(jnp.int32)
    base = idx // L * L
    w = ref[pl.ds(base, L)]
    lane = _splat_i32(idx - base)
    sel = jnp.where(lax.iota(jnp.int32, L) == lane, w, jnp.int32(-2147483648))
    return jnp.max(sel)


def _hsum_splat(acc):
    return jnp.broadcast_to(jnp.sum(acc), (L,))


def _make_readout():
    mesh = plsc.VectorSubcoreMesh(core_axis_name="c", subcore_axis_name="s")

    @functools.partial(
        pl.kernel,
        mesh=mesh,
        compiler_params=pltpu.CompilerParams(needs_layout_passes=False),
        out_type=jax.ShapeDtypeStruct((NG, H), jnp.float32),
        scratch_types=[
            pltpu.VMEM((GPW + L,), jnp.int32),
            pltpu.VMEM((GPW, H), jnp.float32),
            pltpu.VMEM((GPW, H), jnp.float32),
            pltpu.VMEM((2 * T, H), jnp.float32),
            pltpu.VMEM((MAXT,), jnp.int32),
            pltpu.VMEM((MAXT,), jnp.int32),
            pltpu.SemaphoreType.DMA,
            pltpu.SemaphoreType.DMA,
        ],
    )
    def readout(ne_hbm, starts_hbm, h_hbm, jlo_hbm, nseg_hbm, r_hbm,
                starts_v, h_v, r_v, buf, jlo_v, nseg_v, sem0, sem1):
        wid = lax.axis_index("s") * NC + lax.axis_index("c")
        g0 = wid * GPW
        pltpu.sync_copy(starts_hbm.at[pl.ds(g0, GPW + L)], starts_v)
        pltpu.sync_copy(h_hbm.at[pl.ds(g0, GPW)], h_v)
        pltpu.sync_copy(jlo_hbm.at[pl.ds(wid * MAXT, MAXT)], jlo_v)
        pltpu.sync_copy(nseg_hbm.at[pl.ds(wid * MAXT, MAXT)], nseg_v)

        zerov = jnp.zeros((L,), jnp.float32)

        def zero_body(j, _):
            for k in range(KV):
                r_v[j, pl.ds(L * k, L)] = zerov
            return 0

        lax.fori_loop(0, GPW, zero_body, 0)

        LO = _extract_i32(starts_v, 0)
        HI = _extract_i32(starts_v, GPW)
        LO8 = LO // 8 * 8
        nt = (HI - LO8 + (T - 1)) // T
        nt1 = jnp.maximum(nt, 1)
        nt2 = nt1 + (nt1 % 2)
        last = nt2 - 1

        def _fb(t):
            return jnp.minimum(LO8 + t * T, N - T)

        def _start(t, slot_rows, sem):
            pltpu.make_async_copy(
                ne_hbm.at[pl.ds(_fb(t), T)], buf.at[pl.ds(slot_rows, T)], sem
            ).start()

        def _wait(slot_rows, sem):
            pltpu.make_async_copy(
                ne_hbm.at[pl.ds(0, T)], buf.at[pl.ds(slot_rows, T)], sem
            ).wait()

        def process_tile(t, slot, carry):
            tb = LO8 + t * T
            fb = _fb(t)
            jlo = _extract_i32(jlo_v, t)
            njs = _extract_i32(nseg_v, t)
            tb_v = _splat_i32(tb)
            off0 = slot + (tb - fb)
            hi_clamp = slot + (T - 1)

            def seg_body(k, sc):
                m_v, s_v = sc[0], sc[1]
                v_list = list(sc[2:])
                j = jlo + k
                lo = _extract_i32(starts_v, j)
                hi = _extract_i32(starts_v, j + 1)
                hvecs = [h_v[j, pl.ds(L * kk, L)] for kk in range(KV)]
                isnew = _splat_i32(lo) >= tb_v
                m_v = jnp.where(isnew, NEG, m_v)
                s_v = jnp.where(isnew, 0.0, s_v)
                v_list = [jnp.where(isnew, 0.0, vk) for vk in v_list]
                s_lo = jnp.maximum(lo, tb)
                s_hi = jnp.minimum(hi, tb + T)
                ngrp = (s_hi - s_lo + (U - 1)) // U
                off = off0 + (s_lo - tb)
                rend_v = _splat_i32(s_hi)

                def group_body(g, gc):
                    m_v, s_v = gc[0], gc[1]
                    v_list = list(gc[2:])
                    base = off + g * U
                    rows = []
                    es = []
                    for u in range(U):
                        iloc = jnp.minimum(base + u, hi_clamp)
                        rvs = [buf[iloc, pl.ds(L * kk, L)] for kk in range(KV)]
                        acc = rvs[0] * hvecs[0]
                        for kk in range(1, KV):
                            acc = acc + rvs[kk] * hvecs[kk]
                        e_u = _hsum_splat(acc)
                        valid = _splat_i32(s_lo + g * U + u) < rend_v
                        es.append(jnp.where(valid, e_u, NEG))
                        rows.append(rvs)
                    m_new = m_v
                    for u in range(U):
                        m_new = jnp.maximum(m_new, es[u])
                    alpha = jnp.exp(m_v - m_new)
                    ws = [jnp.exp(es[u] - m_new) for u in range(U)]
                    wsum = ws[0]
                    for u in range(1, U):
                        wsum = wsum + ws[u]
                    s_v = s_v * alpha + wsum
                    new_v = []
                    for kk in range(KV):
                        vk = v_list[kk] * alpha
                        for u in range(U):
                            vk = vk + ws[u] * rows[u][kk]
                        new_v.append(vk)
                    return (m_new, s_v, *new_v)

                res = lax.fori_loop(0, ngrp, group_body, (m_v, s_v, *v_list))
                s_fin = res[1]
                denom = s_fin + jnp.float32(1e-16)
                for kk in range(KV):
                    r_v[j, pl.ds(L * kk, L)] = res[2 + kk] / denom
                return res

            return lax.fori_loop(0, njs, seg_body, carry)

        _start(0, 0, sem0)

        def pair_body(i, carry):
            t0 = 2 * i
            t1 = t0 + 1
            _start(jnp.minimum(t1, last), T, sem1)
            _wait(0, sem0)
            carry = process_tile(t0, 0, carry)
            _start(jnp.minimum(t1 + 1, last), 0, sem0)
            _wait(T, sem1)
            carry = process_tile(t1, T, carry)
            return carry

        init = (jnp.full((L,), NEG, jnp.float32), zerov, *([zerov] * KV))
        lax.fori_loop(0, nt2 // 2, pair_body, init)
        _wait(0, sem0)

        pltpu.sync_copy(r_v, r_hbm.at[pl.ds(g0, GPW)])

    return readout


_READOUT = _make_readout()


def _lstm_body(h_ref, r_ref, c_ref, wh_ref, wr_ref, b_ref, h_out, c_out):
    gates = (
        jnp.dot(h_ref[...], wh_ref[...], preferred_element_type=jnp.float32)
        + jnp.dot(r_ref[...], wr_ref[...], preferred_element_type=jnp.float32)
        + b_ref[...]
    )
    i = jax.nn.sigmoid(gates[:, :H])
    f = jax.nn.sigmoid(gates[:, H:2 * H])
    g = jnp.tanh(gates[:, 2 * H:3 * H])
    o = jax.nn.sigmoid(gates[:, 3 * H:4 * H])
    c_new = f * c_ref[...] + i * g
    h_out[...] = o * jnp.tanh(c_new)
    c_out[...] = c_new


_LSTM = pl.pallas_call(
    _lstm_body,
    out_shape=[
        jax.ShapeDtypeStruct((NG, H), jnp.float32),
        jax.ShapeDtypeStruct((NG, H), jnp.float32),
    ],
)


def _final_body(h_ref, r_ref, c_ref, wh_ref, wr_ref, b_ref,
                w1h_ref, w1r_ref, b1_ref, w2t_ref, b2_ref, out_ref):
    gates = (
        jnp.dot(h_ref[...], wh_ref[...], preferred_element_type=jnp.float32)
        + jnp.dot(r_ref[...], wr_ref[...], preferred_element_type=jnp.float32)
        + b_ref[...]
    )
    i = jax.nn.sigmoid(gates[:, :H])
    f = jax.nn.sigmoid(gates[:, H:2 * H])
    g = jnp.tanh(gates[:, 2 * H:3 * H])
    o = jax.nn.sigmoid(gates[:, 3 * H:4 * H])
    c_new = f * c_ref[...] + i * g
    h_new = o * jnp.tanh(c_new)
    hidden = jax.nn.relu(
        jnp.dot(h_new, w1h_ref[...], preferred_element_type=jnp.float32)
        + jnp.dot(r_ref[...], w1r_ref[...], preferred_element_type=jnp.float32)
        + b1_ref[...]
    )
    out_ref[...] = (
        jnp.dot(hidden, w2t_ref[...], preferred_element_type=jnp.float32)
        + b2_ref[...]
    )


_FINAL = pl.pallas_call(
    _final_body,
    out_shape=jax.ShapeDtypeStruct((NG, OUT), jnp.float32),
)


def kernel(node_embeddings, batch_indices, W_ih, W_hh, b_ih, b_hh, W1, b1, W2, b2):
    bi = batch_indices.astype(jnp.int32)
    starts = jnp.searchsorted(bi, jnp.arange(NG + 1, dtype=jnp.int32)).astype(jnp.int32)
    lo_list = starts[:NG].reshape(NW, GPW)
    hi_list = starts[1:NG + 1].reshape(NW, GPW)
    LOs = starts[jnp.arange(NW) * GPW]
    LO8s = LOs // 8 * 8
    tbs = LO8s[:, None] + jnp.arange(MAXT, dtype=jnp.int32)[None, :] * T
    jlo = (hi_list[:, None, :] <= tbs[:, :, None]).sum(-1)
    jhi = (lo_list[:, None, :] < (tbs + T)[:, :, None]).sum(-1)
    nseg = (jhi - jlo).astype(jnp.int32).reshape(-1)
    jlo = jlo.astype(jnp.int32).reshape(-1)
    starts_pad = jnp.concatenate([starts, jnp.full((L - 1,), N, jnp.int32)])

    Wh = (W_ih[:, :H] + W_hh).T
    Wr = W_ih[:, H:].T
    b = (b_ih + b_hh)[None, :]
    W1h = W1[:, :H].T
    W1r = W1[:, H:].T
    b1r = b1[None, :]
    W2T = W2.T
    b2r = b2[None, :]

    h = jnp.zeros((NG, H), jnp.float32)
    c = jnp.zeros((NG, H), jnp.float32)
    out = None
    for step in range(STEPS):
        r = _READOUT(node_embeddings, starts_pad, h, jlo, nseg)
        if step < STEPS - 1:
            h, c = _LSTM(h, r, c, Wh, Wr, b)
        else:
            out = _FINAL(h, r, c, Wh, Wr, b, W1h, W1r, b1r, W2T, b2r)
    return out
